# baseline (speedup 1.0000x reference)
import functools

import jax
import jax.numpy as jnp
import numpy as np
from jax.experimental import pallas as pl
from jax.experimental.pallas import tpu as pltpu

_INTERPRET = False

K = 9
KK = 6 * K
ROT_M = jnp.array([[0.5, -0.5, 0.7071], [0.7071, 0.7071, 0.0], [-0.5, 0.5, 0.7071]],
                  dtype=jnp.float32)


def _xyz2sphere(xyz):
    rho = jnp.sqrt(jnp.sum(xyz ** 2, axis=-1, keepdims=True))
    rho_safe = jnp.where(rho == 0, 1.0, rho)
    theta = jnp.arccos(jnp.clip(xyz[..., 2:3] / rho_safe, -1.0, 1.0))
    phi = jnp.arctan2(xyz[..., 1:2], xyz[..., 0:1])
    theta = jnp.where(rho == 0, 0.0, theta)
    theta = theta / np.pi
    phi = phi / (2 * np.pi) + 0.5
    return jnp.concatenate([rho, theta, phi], axis=-1)


def _front_end(center, offset):
    xyz = center
    N = xyz.shape[0]
    seg = jnp.searchsorted(offset, jnp.arange(N), side='right')
    d2 = (jnp.sum(xyz ** 2, axis=-1, keepdims=True) + jnp.sum(xyz ** 2, axis=-1)[None, :]
          - 2.0 * (xyz @ xyz.T))
    d2 = jnp.where(seg[:, None] != seg[None, :], jnp.inf, d2)
    _, idx = jax.lax.top_k(-d2, KK)
    group_xyz = xyz[idx.reshape(-1)].reshape(N, KK, 3)
    gn = group_xyz - xyz[:, None, :]
    sph = _xyz2sphere(gn @ ROT_M)
    group_r = sph[..., 0]
    sort_idx = jnp.argsort(group_r, axis=-1)
    resort = jnp.take_along_axis(gn, sort_idx[..., None], axis=1)
    parts = [resort[:, :K], resort[:, K:3 * K], resort[:, 3 * K:]]
    outs = []
    for part in parts:
        phi = _xyz2sphere(part @ ROT_M)[..., 2]
        si = jnp.argsort(phi, axis=-1)
        s = jnp.take_along_axis(part, si[..., None], axis=1)[:, :, None, :]
        s_roll = jnp.roll(s, -1, axis=1)
        cen = jnp.zeros_like(s)
        outs.append(jnp.concatenate([cen, s, s_roll], axis=2))
    g = jnp.concatenate(outs, axis=1)

    e1 = g[..., 1, :] - g[..., 0, :]
    e2 = g[..., 2, :] - g[..., 0, :]
    nor = jnp.cross(e1, e2)
    nn = jnp.linalg.norm(nor, axis=-1, keepdims=True)
    n_safe = jnp.where(nn == 0, 1.0, nn)
    unit = nor / n_safe
    pos_mask = (unit[..., 0:1, 0] > 0).astype(jnp.float32) * 2.0 - 1.0
    unit = unit * pos_mask[..., None]
    nan_mask = (nn[..., 0] == 0)

    g_center = jnp.mean(g, axis=-2)
    g_polar = _xyz2sphere(g_center)
    g_pos = jnp.sum(unit * g_center, axis=-1, keepdims=True) / jnp.sqrt(3.0)

    mask_first = jnp.argmax((~nan_mask).astype(jnp.int32), axis=-1)
    rows = jnp.arange(N)
    nf = unit[rows, mask_first][:, None, :]
    cf = g_center[rows, mask_first][:, None, :]
    pf = g_pos[rows, mask_first][:, None, :]
    m = nan_mask[..., None]
    unit = jnp.where(m, nf, unit)
    g_center = jnp.where(m, cf, g_center)
    g_pos = jnp.where(m, pf, g_pos)

    feat = jnp.concatenate([g_polar, unit, g_pos, g_center], axis=-1)
    return feat


def _mlp_kernel(feat_ref, w1s_ref, bias_ref, w2t_ref, b2_ref, out_ref):
    acc = jnp.zeros((feat_ref.shape[0], 64), dtype=jnp.float32)
    w1s = w1s_ref[...]
    bias = bias_ref[...]
    for l in range(KK):
        fl = feat_ref[:, l, :]
        y = jnp.dot(fl, w1s, preferred_element_type=jnp.float32) + bias
        acc = acc + jnp.maximum(y, 0.0)
    out = jnp.dot(acc, w2t_ref[...], preferred_element_type=jnp.float32) + b2_ref[...]
    out_ref[...] = out


def kernel(center, offset, W1, b1, gamma, beta, W2, b2):
    N = center.shape[0]
    feat = _front_end(center, offset)

    L = N * KK
    f2 = feat.reshape(L, 10)
    s = jnp.sum(f2, axis=0)
    M = f2.T @ f2
    w1s_raw = W1 @ s
    mean = w1s_raw / L + b1
    ey2 = jnp.einsum('oc,cd,od->o', W1, M, W1) / L + 2.0 * b1 * (w1s_raw / L) + b1 ** 2
    var = ey2 - mean ** 2
    scale = gamma * jax.lax.rsqrt(var + 1e-5)
    w1s = (W1.T * scale[None, :])
    bias_eff = (b1 - mean) * scale + beta

    featp = jnp.pad(feat, ((0, 0), (0, 0), (0, 6)))
    w1sp = jnp.pad(w1s, ((0, 6), (0, 0)))

    R = 512
    out = pl.pallas_call(
        _mlp_kernel,
        grid=(N // R,),
        in_specs=[
            pl.BlockSpec((R, KK, 16), lambda i: (i, 0, 0)),
            pl.BlockSpec((16, 64), lambda i: (0, 0)),
            pl.BlockSpec((1, 64), lambda i: (0, 0)),
            pl.BlockSpec((64, 64), lambda i: (0, 0)),
            pl.BlockSpec((1, 64), lambda i: (0, 0)),
        ],
        out_specs=pl.BlockSpec((R, 64), lambda i: (i, 0)),
        out_shape=jax.ShapeDtypeStruct((N, 64), jnp.float32),
        interpret=_INTERPRET,
    )(featp, w1sp, bias_eff[None, :], W2.T, (KK * b2)[None, :])
    return out


# speedup vs baseline: 5.1255x; 5.1255x over previous
import functools

import jax
import jax.numpy as jnp
import numpy as np
from jax.experimental import pallas as pl
from jax.experimental.pallas import tpu as pltpu
from jax.experimental.pallas import tpu_sc as plsc

_INTERPRET = False

K = 9
KK = 6 * K
NSEG = 2048
TOPW = 64
PI = float(np.pi)


def _rotl(x, s):
    return jnp.concatenate([x[:, s:], x[:, :s]], axis=1)


def _rotr(x, s):
    return jnp.concatenate([x[:, -s:], x[:, :-s]], axis=1)


def _lane_iota(W):
    return jax.lax.broadcasted_iota(jnp.int32, (1, W), 1)


def _ce(k, i, pay, s, upmask, dirmask):
    pk = jnp.where(upmask, _rotr(k, s), _rotl(k, s))
    pi = jnp.where(upmask, _rotr(i, s), _rotl(i, s))
    cmp = (k > pk) | ((k == pk) & (i > pi))
    take = cmp ^ upmask ^ dirmask
    k = jnp.where(take, pk, k)
    i = jnp.where(take, pi, i)
    pay = [jnp.where(take, jnp.where(upmask, _rotr(p, s), _rotl(p, s)), p)
           for p in pay]
    return k, i, pay


def _sort_blocks(k, i, pay, bsz, final_dir):
    W = k.shape[1]
    io = _lane_iota(W)
    dirs = {bsz: final_dir}
    m = bsz // 2
    while m >= 2:
        dirs[m] = dirs[2 * m] ^ ((io & m) != 0)
        m //= 2
    m = 2
    while m <= bsz:
        d = dirs[m]
        s = m // 2
        while s >= 1:
            up = (io & s) != 0
            k, i, pay = _ce(k, i, pay, s, up, d)
            s //= 2
        m *= 2
    return k, i, pay


def _merge_cleanup(k, i, pay, dirmask):
    for s in (32, 16, 8, 4, 2, 1):
        up = (_lane_iota(k.shape[1]) & s) != 0
        k, i, pay = _ce(k, i, pay, s, up, dirmask)
    return k, i, pay


def _top64(k, i):
    W = k.shape[1]
    fd = _lane_iota(W) >= (W // 2)
    k, i, _ = _sort_blocks(k, i, [], 64, fd)
    while W > TOPW:
        H = W // 2
        ka, kb = k[:, :H], k[:, H:]
        ia, ib = i[:, :H], i[:, H:]
        cmp = (ka > kb) | ((ka == kb) & (ia > ib))
        k = jnp.where(cmp, kb, ka)
        i = jnp.where(cmp, ib, ia)
        W = H
        if W > TOPW:
            d = _lane_iota(W) >= (W // 2)
        else:
            d = jnp.zeros((1, W), dtype=bool)
        k, i, _ = _merge_cleanup(k, i, [], d)
    return k, i


def _knn_body(q_ref, c_ref, idx_ref, *, rblk):
    q = q_ref[...]
    c = c_ref[...]
    qs = jnp.sum(q * q, axis=1, keepdims=True)
    cs = jnp.sum(c * c, axis=1, keepdims=True)
    g = jnp.dot(q.astype(jnp.bfloat16), c.T.astype(jnp.bfloat16),
                preferred_element_type=jnp.float32)
    d2 = qs + cs.T - 2.0 * g
    ii = jax.lax.broadcasted_iota(jnp.int32, (rblk, NSEG), 1)
    _, itop = _top64(d2, ii)
    idx_ref[...] = itop + pl.program_id(0) * NSEG


def _knn_top64(center, rblk=256):
    N = center.shape[0]
    nseg = N // NSEG
    nr = NSEG // rblk
    return pl.pallas_call(
        functools.partial(_knn_body, rblk=rblk),
        grid=(nseg, nr),
        in_specs=[
            pl.BlockSpec((rblk, 3), lambda s, r: (s * nr + r, 0)),
            pl.BlockSpec((NSEG, 3), lambda s, r: (s, 0)),
        ],
        out_specs=pl.BlockSpec((rblk, TOPW), lambda s, r: (s * nr + r, 0)),
        out_shape=jax.ShapeDtypeStruct((N, TOPW), jnp.int32),
        interpret=_INTERPRET,
    )(center, center)


def _sc_gather(xt, yt, zt, idx2d):
    NR, RW = idx2d.shape
    NW = 32
    rpw = NR // NW
    mesh = plsc.VectorSubcoreMesh(core_axis_name="c", subcore_axis_name="s")
    o = jax.ShapeDtypeStruct((NR, RW), jnp.float32)

    @functools.partial(
        pl.kernel, mesh=mesh,
        out_type=[o, o, o],
        scratch_types=[
            pltpu.VMEM((rpw, RW), jnp.int32),
            pltpu.VMEM((rpw, RW), jnp.float32),
            pltpu.VMEM((rpw, RW), jnp.float32),
            pltpu.VMEM((rpw, RW), jnp.float32),
            pltpu.SemaphoreType.DMA,
        ],
    )
    def gk(xt_h, yt_h, zt_h, idx_h, xo_h, yo_h, zo_h,
           idx_v, xb, yb, zb, sem):
        wid = jax.lax.axis_index("s") * 2 + jax.lax.axis_index("c")
        base = wid * rpw
        pltpu.sync_copy(idx_h.at[pl.ds(base, rpw)], idx_v)

        def body(j, _):
            row = idx_v.at[j]
            pltpu.async_copy(xt_h.at[row], xb.at[j], sem).wait()
            pltpu.async_copy(yt_h.at[row], yb.at[j], sem).wait()
            pltpu.async_copy(zt_h.at[row], zb.at[j], sem).wait()
            return 0

        jax.lax.fori_loop(0, rpw, body, 0)
        pltpu.sync_copy(xb, xo_h.at[pl.ds(base, rpw)])
        pltpu.sync_copy(yb, yo_h.at[pl.ds(base, rpw)])
        pltpu.sync_copy(zb, zo_h.at[pl.ds(base, rpw)])

    return gk(xt, yt, zt, idx2d)


_C = np.float32(0.7071)
_CB = np.float32(np.array(0.7071, dtype=jnp.bfloat16))


def _bf(x):
    return x.astype(jnp.bfloat16).astype(jnp.float32)


def _rot3k(dx, dy, dz):
    dx, dy, dz = _bf(dx), _bf(dy), _bf(dz)
    rx = (0.5 * dx + _CB * dy) - 0.5 * dz
    ry = (-0.5 * dx + _CB * dy) + 0.5 * dz
    rz = _CB * dx + _CB * dz
    return rx, ry, rz


def _atan2(y, x):
    ax, ay = jnp.abs(x), jnp.abs(y)
    sw = ay > ax
    num = jnp.where(sw, ax, ay)
    den = jnp.where(sw, ay, ax)
    r = num / jnp.where(den == 0, 1.0, den)
    z = r * r
    a = ((((0.0805374449538 * z - 0.138776856032) * z + 0.199777106478) * z
          - 0.333329491539) * z) * r + r
    a = jnp.where(sw, np.float32(PI / 2) - a, a)
    xneg = jax.lax.bitcast_convert_type(x, jnp.int32) < 0
    a = jnp.where(xneg, np.float32(PI) - a, a)
    yneg = jax.lax.bitcast_convert_type(y, jnp.int32) < 0
    return jnp.where(yneg, -a, a)


def _pseudo_angle(ry, rx):
    sab = jnp.abs(rx) + jnp.abs(ry)
    core = 1.0 - rx / jnp.where(sab == 0, 1.0, sab)
    yneg = jax.lax.bitcast_convert_type(ry, jnp.int32) < 0
    p = jnp.where(yneg, -core, core)
    return jnp.where(sab == 0, 0.0, p)


def _geom_body(xg_ref, yg_ref, zg_ref, c_ref, feat_ref, mom_ref, *, rblk):
    R = rblk
    INF = jnp.float32(np.inf)
    lane = jax.lax.broadcasted_iota(jnp.int32, (1, TOPW), 1)
    dx = xg_ref[...] - c_ref[:, 0:1]
    dy = yg_ref[...] - c_ref[:, 1:2]
    dz = zg_ref[...] - c_ref[:, 2:3]
    rx, ry, rz = _rot3k(dx, dy, dz)
    rho = jnp.sqrt(rx * rx + ry * ry + rz * rz)
    key = jnp.where(lane < KK, rho, INF)
    pos = jax.lax.broadcasted_iota(jnp.int32, (R, TOPW), 1)
    asc = jnp.zeros((1, TOPW), dtype=bool)
    _, _, (dx, dy, dz) = _sort_blocks(key, pos, [dx, dy, dz], TOPW, asc)

    sparts, nparts = [], []
    for off, m, M in ((0, 9, 16), (9, 18, 32), (27, 27, 32)):
        pads = M - m
        px = jnp.concatenate([dx[:, off:off + m], jnp.zeros((R, pads), jnp.float32)], 1)
        py = jnp.concatenate([dy[:, off:off + m], jnp.zeros((R, pads), jnp.float32)], 1)
        pz = jnp.concatenate([dz[:, off:off + m], jnp.zeros((R, pads), jnp.float32)], 1)
        prx, pry, _ = _rot3k(px, py, pz)
        pa = _pseudo_angle(pry, prx)
        pl_lane = _lane_iota(M)
        k2 = jnp.where(pl_lane < m, pa, INF)
        p2 = jax.lax.broadcasted_iota(jnp.int32, (R, M), 1)
        ascM = jnp.zeros((1, M), dtype=bool)
        _, _, (px, py, pz) = _sort_blocks(k2, p2, [px, py, pz], M, ascM)
        wrap = pl_lane == (m - 1)
        nx = jnp.where(wrap, px[:, 0:1], _rotl(px, 1))
        ny = jnp.where(wrap, py[:, 0:1], _rotl(py, 1))
        nz = jnp.where(wrap, pz[:, 0:1], _rotl(pz, 1))
        sparts.append((px[:, :m], py[:, :m], pz[:, :m]))
        nparts.append((nx[:, :m], ny[:, :m], nz[:, :m]))

    zpad = jnp.zeros((R, TOPW - KK), jnp.float32)
    sdx = jnp.concatenate([sparts[0][0], sparts[1][0], sparts[2][0], zpad], 1)
    sdy = jnp.concatenate([sparts[0][1], sparts[1][1], sparts[2][1], zpad], 1)
    sdz = jnp.concatenate([sparts[0][2], sparts[1][2], sparts[2][2], zpad], 1)
    ndx = jnp.concatenate([nparts[0][0], nparts[1][0], nparts[2][0], zpad], 1)
    ndy = jnp.concatenate([nparts[0][1], nparts[1][1], nparts[2][1], zpad], 1)
    ndz = jnp.concatenate([nparts[0][2], nparts[1][2], nparts[2][2], zpad], 1)

    nx = sdy * ndz - sdz * ndy
    ny = sdz * ndx - sdx * ndz
    nz = sdx * ndy - sdy * ndx
    nn = jnp.sqrt(nx * nx + ny * ny + nz * nz)
    bad = nn == 0
    inv = 1.0 / jnp.where(bad, 1.0, nn)
    ux, uy, uz = nx * inv, ny * inv, nz * inv
    pm = jnp.where(ux[:, 0:1] > 0, 1.0, -1.0).astype(jnp.float32)
    ux, uy, uz = ux * pm, uy * pm, uz * pm

    gcx = (sdx + ndx) / 3.0
    gcy = (sdy + ndy) / 3.0
    gcz = (sdz + ndz) / 3.0
    rc = jnp.sqrt(gcx * gcx + gcy * gcy + gcz * gcz)
    rsafe = jnp.where(rc == 0, 1.0, rc)
    tt = jnp.clip(gcz / rsafe, -1.0, 1.0)
    theta = _atan2(jnp.sqrt(jnp.maximum(1.0 - tt * tt, 0.0)), tt)
    theta = jnp.where(rc == 0, 0.0, theta) * np.float32(1.0 / PI)
    phic = _atan2(gcy, gcx) * np.float32(1.0 / (2 * PI)) + 0.5
    gpos = (ux * gcx + uy * gcy + uz * gcz) / np.float32(np.sqrt(3.0))

    cand = jnp.where((~bad) & (lane < KK), pos, 999)
    fi = jnp.min(cand, axis=1, keepdims=True)
    fi = jnp.where(fi == 999, 0, fi)
    sel = (pos == fi).astype(jnp.float32)

    def firstv(v):
        return jnp.sum(v * sel, axis=1, keepdims=True)

    ux = jnp.where(bad, firstv(ux), ux)
    uy = jnp.where(bad, firstv(uy), uy)
    uz = jnp.where(bad, firstv(uz), uz)
    gpos = jnp.where(bad, firstv(gpos), gpos)
    gcxf = jnp.where(bad, firstv(gcx), gcx)
    gcyf = jnp.where(bad, firstv(gcy), gcy)
    gczf = jnp.where(bad, firstv(gcz), gcz)

    valid = lane < KK
    feats = [rc, theta, phic, ux, uy, uz, gpos, gcxf, gcyf, gczf]
    feats = [jnp.where(valid, f, 0.0) for f in feats]
    for cix in range(10):
        feat_ref[cix] = feats[cix]

    rio = jax.lax.broadcasted_iota(jnp.int32, (16, 128), 0)
    cio = jax.lax.broadcasted_iota(jnp.int32, (16, 128), 1)
    mp = jnp.zeros((16, 128), jnp.float32)
    for c in range(10):
        sc_ = jnp.sum(feats[c])
        mp = mp + jnp.where((rio == 10) & (cio == c), sc_, 0.0)
        for d in range(c, 10):
            v = jnp.sum(feats[c] * feats[d])
            msk = (rio == c) & (cio == d)
            if d != c:
                msk = msk | ((rio == d) & (cio == c))
            mp = mp + jnp.where(msk, v, 0.0)

    @pl.when(pl.program_id(0) == 0)
    def _():
        mom_ref[...] = jnp.zeros((16, 128), jnp.float32)

    mom_ref[...] += mp


def _geometry(xg, yg, zg, center, rblk=256):
    N = center.shape[0]
    return pl.pallas_call(
        functools.partial(_geom_body, rblk=rblk),
        grid=(N // rblk,),
        in_specs=[
            pl.BlockSpec((rblk, TOPW), lambda i: (i, 0)),
            pl.BlockSpec((rblk, TOPW), lambda i: (i, 0)),
            pl.BlockSpec((rblk, TOPW), lambda i: (i, 0)),
            pl.BlockSpec((rblk, 3), lambda i: (i, 0)),
        ],
        out_specs=[
            pl.BlockSpec((10, rblk, TOPW), lambda i: (0, i, 0)),
            pl.BlockSpec((16, 128), lambda i: (0, 0)),
        ],
        out_shape=[
            jax.ShapeDtypeStruct((10, N, TOPW), jnp.float32),
            jax.ShapeDtypeStruct((16, 128), jnp.float32),
        ],
        interpret=_INTERPRET,
    )(xg, yg, zg, center)


def _mlp_body(f_ref, w1_ref, b_ref, w2_ref, b2_ref, out_ref, *, rblk):
    R = rblk
    lane = jax.lax.broadcasted_iota(jnp.int32, (1, TOPW), 1)
    valid = lane < KK
    colio = jax.lax.broadcasted_iota(jnp.int32, (1, 64), 1)
    F = [f_ref[c] for c in range(10)]
    z = jnp.zeros((R, 64), jnp.float32)
    for o in range(64):
        y = jnp.full((R, TOPW), b_ref[0, o], jnp.float32)
        for c in range(10):
            y = y + w1_ref[c, o] * F[c]
        y = jnp.maximum(y, 0.0)
        zo = jnp.sum(jnp.where(valid, y, 0.0), axis=1, keepdims=True)
        z = z + jnp.where(colio == o, zo, 0.0)
    out_ref[...] = (jnp.dot(z, w2_ref[...], preferred_element_type=jnp.float32)
                    + b2_ref[...])


def _mlp(f_all, w1s, bias_eff, W2T, b2eff, rblk=256):
    N = f_all.shape[1]
    return pl.pallas_call(
        functools.partial(_mlp_body, rblk=rblk),
        grid=(N // rblk,),
        in_specs=[
            pl.BlockSpec((10, rblk, TOPW), lambda i: (0, i, 0)),
            pl.BlockSpec((16, 64), lambda i: (0, 0)),
            pl.BlockSpec((1, 64), lambda i: (0, 0)),
            pl.BlockSpec((64, 64), lambda i: (0, 0)),
            pl.BlockSpec((1, 64), lambda i: (0, 0)),
        ],
        out_specs=pl.BlockSpec((rblk, 64), lambda i: (i, 0)),
        out_shape=jax.ShapeDtypeStruct((N, 64), jnp.float32),
        interpret=_INTERPRET,
    )(f_all, w1s, bias_eff, W2T, b2eff)


def _gather_xyz(center, idx):
    N = center.shape[0]
    xt = center[:, 0]
    yt = center[:, 1]
    zt = center[:, 2]
    xg, yg, zg = _sc_gather(xt, yt, zt, idx.reshape(-1, 128))
    return xg.reshape(N, TOPW), yg.reshape(N, TOPW), zg.reshape(N, TOPW)


def kernel(center, offset, W1, b1, gamma, beta, W2, b2):
    N = center.shape[0]
    idx = _knn_top64(center)
    xg, yg, zg = _gather_xyz(center, idx)
    f_all, mom = _geometry(xg, yg, zg, center)

    L = N * KK
    Mm = mom[:10, :10]
    s = mom[10, :10]
    w1s_raw = W1 @ s
    mean = w1s_raw / L + b1
    ey2 = jnp.einsum('oc,cd,od->o', W1, Mm, W1) / L + 2.0 * b1 * (w1s_raw / L) + b1 ** 2
    var = ey2 - mean ** 2
    scale = gamma * jax.lax.rsqrt(var + 1e-5)
    w1s = jnp.pad(W1.T * scale[None, :], ((0, 6), (0, 0)))
    bias_eff = ((b1 - mean) * scale + beta)[None, :]
    return _mlp(f_all, w1s, bias_eff, W2.T, (KK * b2)[None, :])


# speedup vs baseline: 5.2847x; 1.0311x over previous
import functools

import jax
import jax.numpy as jnp
import numpy as np
from jax.experimental import pallas as pl
from jax.experimental.pallas import tpu as pltpu
from jax.experimental.pallas import tpu_sc as plsc

_INTERPRET = False

K = 9
KK = 6 * K
NSEG = 2048
TOPW = 64
PI = float(np.pi)


def _rotl(x, s):
    return jnp.concatenate([x[:, s:], x[:, :s]], axis=1)


def _rotr(x, s):
    return jnp.concatenate([x[:, -s:], x[:, :-s]], axis=1)


def _lane_iota(W):
    return jax.lax.broadcasted_iota(jnp.int32, (1, W), 1)


def _ce(k, i, pay, s, upmask, dirmask):
    pk = jnp.where(upmask, _rotr(k, s), _rotl(k, s))
    pi = jnp.where(upmask, _rotr(i, s), _rotl(i, s))
    cmp = (k > pk) | ((k == pk) & (i > pi))
    take = cmp ^ upmask ^ dirmask
    k = jnp.where(take, pk, k)
    i = jnp.where(take, pi, i)
    pay = [jnp.where(take, jnp.where(upmask, _rotr(p, s), _rotl(p, s)), p)
           for p in pay]
    return k, i, pay


def _sort_blocks(k, i, pay, bsz, final_dir):
    W = k.shape[1]
    io = _lane_iota(W)
    dirs = {bsz: final_dir}
    m = bsz // 2
    while m >= 2:
        dirs[m] = dirs[2 * m] ^ ((io & m) != 0)
        m //= 2
    m = 2
    while m <= bsz:
        d = dirs[m]
        s = m // 2
        while s >= 1:
            up = (io & s) != 0
            k, i, pay = _ce(k, i, pay, s, up, d)
            s //= 2
        m *= 2
    return k, i, pay


def _merge_cleanup(k, i, pay, dirmask):
    for s in (32, 16, 8, 4, 2, 1):
        up = (_lane_iota(k.shape[1]) & s) != 0
        k, i, pay = _ce(k, i, pay, s, up, dirmask)
    return k, i, pay


def _top64(k, i):
    W = k.shape[1]
    fd = _lane_iota(W) >= (W // 2)
    k, i, _ = _sort_blocks(k, i, [], 64, fd)
    while W > TOPW:
        H = W // 2
        ka, kb = k[:, :H], k[:, H:]
        ia, ib = i[:, :H], i[:, H:]
        cmp = (ka > kb) | ((ka == kb) & (ia > ib))
        k = jnp.where(cmp, kb, ka)
        i = jnp.where(cmp, ib, ia)
        W = H
        if W > TOPW:
            d = _lane_iota(W) >= (W // 2)
        else:
            d = jnp.zeros((1, W), dtype=bool)
        k, i, _ = _merge_cleanup(k, i, [], d)
    return k, i


def _knn_body(q_ref, c_ref, idx_ref, *, rblk):
    q = q_ref[...]
    c = c_ref[...]
    qs = jnp.sum(q * q, axis=1, keepdims=True)
    cs = jnp.sum(c * c, axis=1, keepdims=True)
    g = jnp.dot(q.astype(jnp.bfloat16), c.T.astype(jnp.bfloat16),
                preferred_element_type=jnp.float32)
    d2 = qs + cs.T - 2.0 * g
    ii = jax.lax.broadcasted_iota(jnp.int32, (rblk, NSEG), 1)
    _, itop = _top64(d2, ii)
    idx_ref[...] = itop + pl.program_id(0) * NSEG


def _knn_top64(center, rblk=512):
    N = center.shape[0]
    nseg = N // NSEG
    nr = NSEG // rblk
    return pl.pallas_call(
        functools.partial(_knn_body, rblk=rblk),
        grid=(nseg, nr),
        in_specs=[
            pl.BlockSpec((rblk, 3), lambda s, r: (s * nr + r, 0)),
            pl.BlockSpec((NSEG, 3), lambda s, r: (s, 0)),
        ],
        out_specs=pl.BlockSpec((rblk, TOPW), lambda s, r: (s * nr + r, 0)),
        out_shape=jax.ShapeDtypeStruct((N, TOPW), jnp.int32),
        interpret=_INTERPRET,
    )(center, center)


def _sc_gather(xt, yt, zt, idx2d):
    NR, RW = idx2d.shape
    NW = 32
    rpw = NR // NW
    mesh = plsc.VectorSubcoreMesh(core_axis_name="c", subcore_axis_name="s")
    o = jax.ShapeDtypeStruct((NR, RW), jnp.float32)

    @functools.partial(
        pl.kernel, mesh=mesh,
        out_type=[o, o, o],
        scratch_types=[
            pltpu.VMEM((rpw, RW), jnp.int32),
            pltpu.VMEM((rpw, RW), jnp.float32),
            pltpu.VMEM((rpw, RW), jnp.float32),
            pltpu.VMEM((rpw, RW), jnp.float32),
            pltpu.SemaphoreType.DMA,
        ],
    )
    def gk(xt_h, yt_h, zt_h, idx_h, xo_h, yo_h, zo_h,
           idx_v, xb, yb, zb, sem):
        wid = jax.lax.axis_index("s") * 2 + jax.lax.axis_index("c")
        base = wid * rpw
        pltpu.sync_copy(idx_h.at[pl.ds(base, rpw)], idx_v)

        def body(j, _):
            row = idx_v.at[j]
            pltpu.async_copy(xt_h.at[row], xb.at[j], sem).wait()
            pltpu.async_copy(yt_h.at[row], yb.at[j], sem).wait()
            pltpu.async_copy(zt_h.at[row], zb.at[j], sem).wait()
            return 0

        jax.lax.fori_loop(0, rpw, body, 0)
        pltpu.sync_copy(xb, xo_h.at[pl.ds(base, rpw)])
        pltpu.sync_copy(yb, yo_h.at[pl.ds(base, rpw)])
        pltpu.sync_copy(zb, zo_h.at[pl.ds(base, rpw)])

    return gk(xt, yt, zt, idx2d)


_C = np.float32(0.7071)
_CB = np.float32(np.array(0.7071, dtype=jnp.bfloat16))


def _bf(x):
    return x.astype(jnp.bfloat16).astype(jnp.float32)


def _rot3k(dx, dy, dz):
    dx, dy, dz = _bf(dx), _bf(dy), _bf(dz)
    rx = (0.5 * dx + _CB * dy) - 0.5 * dz
    ry = (-0.5 * dx + _CB * dy) + 0.5 * dz
    rz = _CB * dx + _CB * dz
    return rx, ry, rz


def _atan2(y, x):
    ax, ay = jnp.abs(x), jnp.abs(y)
    sw = ay > ax
    num = jnp.where(sw, ax, ay)
    den = jnp.where(sw, ay, ax)
    r = num / jnp.where(den == 0, 1.0, den)
    z = r * r
    a = ((((0.0805374449538 * z - 0.138776856032) * z + 0.199777106478) * z
          - 0.333329491539) * z) * r + r
    a = jnp.where(sw, np.float32(PI / 2) - a, a)
    xneg = jax.lax.bitcast_convert_type(x, jnp.int32) < 0
    a = jnp.where(xneg, np.float32(PI) - a, a)
    yneg = jax.lax.bitcast_convert_type(y, jnp.int32) < 0
    return jnp.where(yneg, -a, a)


def _pseudo_angle(ry, rx):
    sab = jnp.abs(rx) + jnp.abs(ry)
    core = 1.0 - rx / jnp.where(sab == 0, 1.0, sab)
    yneg = jax.lax.bitcast_convert_type(ry, jnp.int32) < 0
    p = jnp.where(yneg, -core, core)
    return jnp.where(sab == 0, 0.0, p)


def _geom_body(xg_ref, yg_ref, zg_ref, c_ref, feat_ref, mom_ref, *, rblk):
    R = rblk
    INF = jnp.float32(np.inf)
    lane = jax.lax.broadcasted_iota(jnp.int32, (1, TOPW), 1)
    dx = xg_ref[...] - c_ref[:, 0:1]
    dy = yg_ref[...] - c_ref[:, 1:2]
    dz = zg_ref[...] - c_ref[:, 2:3]
    rx, ry, rz = _rot3k(dx, dy, dz)
    rho = jnp.sqrt(rx * rx + ry * ry + rz * rz)
    key = jnp.where(lane < KK, rho, INF)
    pos = jax.lax.broadcasted_iota(jnp.int32, (R, TOPW), 1)
    asc = jnp.zeros((1, TOPW), dtype=bool)
    _, _, (dx, dy, dz) = _sort_blocks(key, pos, [dx, dy, dz], TOPW, asc)

    sparts, nparts = [], []
    for off, m, M in ((0, 9, 16), (9, 18, 32), (27, 27, 32)):
        pads = M - m
        px = jnp.concatenate([dx[:, off:off + m], jnp.zeros((R, pads), jnp.float32)], 1)
        py = jnp.concatenate([dy[:, off:off + m], jnp.zeros((R, pads), jnp.float32)], 1)
        pz = jnp.concatenate([dz[:, off:off + m], jnp.zeros((R, pads), jnp.float32)], 1)
        prx, pry, _ = _rot3k(px, py, pz)
        pa = _pseudo_angle(pry, prx)
        pl_lane = _lane_iota(M)
        k2 = jnp.where(pl_lane < m, pa, INF)
        p2 = jax.lax.broadcasted_iota(jnp.int32, (R, M), 1)
        ascM = jnp.zeros((1, M), dtype=bool)
        _, _, (px, py, pz) = _sort_blocks(k2, p2, [px, py, pz], M, ascM)
        wrap = pl_lane == (m - 1)
        nx = jnp.where(wrap, px[:, 0:1], _rotl(px, 1))
        ny = jnp.where(wrap, py[:, 0:1], _rotl(py, 1))
        nz = jnp.where(wrap, pz[:, 0:1], _rotl(pz, 1))
        sparts.append((px[:, :m], py[:, :m], pz[:, :m]))
        nparts.append((nx[:, :m], ny[:, :m], nz[:, :m]))

    zpad = jnp.zeros((R, TOPW - KK), jnp.float32)
    sdx = jnp.concatenate([sparts[0][0], sparts[1][0], sparts[2][0], zpad], 1)
    sdy = jnp.concatenate([sparts[0][1], sparts[1][1], sparts[2][1], zpad], 1)
    sdz = jnp.concatenate([sparts[0][2], sparts[1][2], sparts[2][2], zpad], 1)
    ndx = jnp.concatenate([nparts[0][0], nparts[1][0], nparts[2][0], zpad], 1)
    ndy = jnp.concatenate([nparts[0][1], nparts[1][1], nparts[2][1], zpad], 1)
    ndz = jnp.concatenate([nparts[0][2], nparts[1][2], nparts[2][2], zpad], 1)

    nx = sdy * ndz - sdz * ndy
    ny = sdz * ndx - sdx * ndz
    nz = sdx * ndy - sdy * ndx
    nn = jnp.sqrt(nx * nx + ny * ny + nz * nz)
    bad = nn == 0
    inv = 1.0 / jnp.where(bad, 1.0, nn)
    ux, uy, uz = nx * inv, ny * inv, nz * inv
    pm = jnp.where(ux[:, 0:1] > 0, 1.0, -1.0).astype(jnp.float32)
    ux, uy, uz = ux * pm, uy * pm, uz * pm

    gcx = (sdx + ndx) / 3.0
    gcy = (sdy + ndy) / 3.0
    gcz = (sdz + ndz) / 3.0
    rc = jnp.sqrt(gcx * gcx + gcy * gcy + gcz * gcz)
    rsafe = jnp.where(rc == 0, 1.0, rc)
    tt = jnp.clip(gcz / rsafe, -1.0, 1.0)
    theta = _atan2(jnp.sqrt(jnp.maximum(1.0 - tt * tt, 0.0)), tt)
    theta = jnp.where(rc == 0, 0.0, theta) * np.float32(1.0 / PI)
    phic = _atan2(gcy, gcx) * np.float32(1.0 / (2 * PI)) + 0.5
    gpos = (ux * gcx + uy * gcy + uz * gcz) / np.float32(np.sqrt(3.0))

    cand = jnp.where((~bad) & (lane < KK), pos, 999)
    fi = jnp.min(cand, axis=1, keepdims=True)
    fi = jnp.where(fi == 999, 0, fi)
    sel = (pos == fi).astype(jnp.float32)

    def firstv(v):
        return jnp.sum(v * sel, axis=1, keepdims=True)

    ux = jnp.where(bad, firstv(ux), ux)
    uy = jnp.where(bad, firstv(uy), uy)
    uz = jnp.where(bad, firstv(uz), uz)
    gpos = jnp.where(bad, firstv(gpos), gpos)
    gcxf = jnp.where(bad, firstv(gcx), gcx)
    gcyf = jnp.where(bad, firstv(gcy), gcy)
    gczf = jnp.where(bad, firstv(gcz), gcz)

    valid = lane < KK
    feats = [rc, theta, phic, ux, uy, uz, gpos, gcxf, gcyf, gczf]
    feats = [jnp.where(valid, f, 0.0) for f in feats]
    for cix in range(10):
        feat_ref[cix] = feats[cix]

    rio = jax.lax.broadcasted_iota(jnp.int32, (16, 128), 0)
    cio = jax.lax.broadcasted_iota(jnp.int32, (16, 128), 1)
    mp = jnp.zeros((16, 128), jnp.float32)
    for c in range(10):
        sc_ = jnp.sum(feats[c])
        mp = mp + jnp.where((rio == 10) & (cio == c), sc_, 0.0)
        for d in range(c, 10):
            v = jnp.sum(feats[c] * feats[d])
            msk = (rio == c) & (cio == d)
            if d != c:
                msk = msk | ((rio == d) & (cio == c))
            mp = mp + jnp.where(msk, v, 0.0)

    @pl.when(pl.program_id(0) == 0)
    def _():
        mom_ref[...] = jnp.zeros((16, 128), jnp.float32)

    mom_ref[...] += mp


def _geometry(xg, yg, zg, center, rblk=256):
    N = center.shape[0]
    return pl.pallas_call(
        functools.partial(_geom_body, rblk=rblk),
        grid=(N // rblk,),
        in_specs=[
            pl.BlockSpec((rblk, TOPW), lambda i: (i, 0)),
            pl.BlockSpec((rblk, TOPW), lambda i: (i, 0)),
            pl.BlockSpec((rblk, TOPW), lambda i: (i, 0)),
            pl.BlockSpec((rblk, 3), lambda i: (i, 0)),
        ],
        out_specs=[
            pl.BlockSpec((10, rblk, TOPW), lambda i: (0, i, 0)),
            pl.BlockSpec((16, 128), lambda i: (0, 0)),
        ],
        out_shape=[
            jax.ShapeDtypeStruct((10, N, TOPW), jnp.float32),
            jax.ShapeDtypeStruct((16, 128), jnp.float32),
        ],
        interpret=_INTERPRET,
    )(xg, yg, zg, center)


def _mlp_body(f_ref, w1_ref, b_ref, w2_ref, b2_ref, out_ref, *, rblk):
    R = rblk
    lane = jax.lax.broadcasted_iota(jnp.int32, (1, TOPW), 1)
    valid = lane < KK
    colio = jax.lax.broadcasted_iota(jnp.int32, (1, 64), 1)
    F = [f_ref[c] for c in range(10)]
    z = jnp.zeros((R, 64), jnp.float32)
    for o in range(64):
        y = jnp.full((R, TOPW), b_ref[0, o], jnp.float32)
        for c in range(10):
            y = y + w1_ref[c, o] * F[c]
        y = jnp.maximum(y, 0.0)
        zo = jnp.sum(jnp.where(valid, y, 0.0), axis=1, keepdims=True)
        z = z + jnp.where(colio == o, zo, 0.0)
    out_ref[...] = (jnp.dot(z, w2_ref[...], preferred_element_type=jnp.float32)
                    + b2_ref[...])


def _mlp(f_all, w1s, bias_eff, W2T, b2eff, rblk=256):
    N = f_all.shape[1]
    return pl.pallas_call(
        functools.partial(_mlp_body, rblk=rblk),
        grid=(N // rblk,),
        in_specs=[
            pl.BlockSpec((10, rblk, TOPW), lambda i: (0, i, 0)),
            pl.BlockSpec((16, 64), lambda i: (0, 0)),
            pl.BlockSpec((1, 64), lambda i: (0, 0)),
            pl.BlockSpec((64, 64), lambda i: (0, 0)),
            pl.BlockSpec((1, 64), lambda i: (0, 0)),
        ],
        out_specs=pl.BlockSpec((rblk, 64), lambda i: (i, 0)),
        out_shape=jax.ShapeDtypeStruct((N, 64), jnp.float32),
        interpret=_INTERPRET,
    )(f_all, w1s, bias_eff, W2T, b2eff)


def _gather_xyz(center, idx):
    N = center.shape[0]
    xt = center[:, 0]
    yt = center[:, 1]
    zt = center[:, 2]
    xg, yg, zg = _sc_gather(xt, yt, zt, idx.reshape(-1, 128))
    return xg.reshape(N, TOPW), yg.reshape(N, TOPW), zg.reshape(N, TOPW)


def kernel(center, offset, W1, b1, gamma, beta, W2, b2):
    N = center.shape[0]
    idx = _knn_top64(center)
    xg, yg, zg = _gather_xyz(center, idx)
    f_all, mom = _geometry(xg, yg, zg, center)

    L = N * KK
    Mm = mom[:10, :10]
    s = mom[10, :10]
    w1s_raw = W1 @ s
    mean = w1s_raw / L + b1
    ey2 = jnp.einsum('oc,cd,od->o', W1, Mm, W1) / L + 2.0 * b1 * (w1s_raw / L) + b1 ** 2
    var = ey2 - mean ** 2
    scale = gamma * jax.lax.rsqrt(var + 1e-5)
    w1s = jnp.pad(W1.T * scale[None, :], ((0, 6), (0, 0)))
    bias_eff = ((b1 - mean) * scale + beta)[None, :]
    return _mlp(f_all, w1s, bias_eff, W2.T, (KK * b2)[None, :])


# speedup vs baseline: 5.3234x; 1.0073x over previous
import functools

import jax
import jax.numpy as jnp
import numpy as np
from jax.experimental import pallas as pl
from jax.experimental.pallas import tpu as pltpu
from jax.experimental.pallas import tpu_sc as plsc

_INTERPRET = False

K = 9
KK = 6 * K
NSEG = 2048
TOPW = 64
PI = float(np.pi)


def _rotl(x, s):
    return jnp.concatenate([x[:, s:], x[:, :s]], axis=1)


def _rotr(x, s):
    return jnp.concatenate([x[:, -s:], x[:, :-s]], axis=1)


def _lane_iota(W):
    return jax.lax.broadcasted_iota(jnp.int32, (1, W), 1)


def _ce(k, i, pay, s, upmask, dirmask):
    pk = jnp.where(upmask, _rotr(k, s), _rotl(k, s))
    pi = jnp.where(upmask, _rotr(i, s), _rotl(i, s))
    cmp = (k > pk) | ((k == pk) & (i > pi))
    take = cmp ^ upmask ^ dirmask
    k = jnp.where(take, pk, k)
    i = jnp.where(take, pi, i)
    pay = [jnp.where(take, jnp.where(upmask, _rotr(p, s), _rotl(p, s)), p)
           for p in pay]
    return k, i, pay


def _sort_blocks(k, i, pay, bsz, final_dir):
    W = k.shape[1]
    io = _lane_iota(W)
    dirs = {bsz: final_dir}
    m = bsz // 2
    while m >= 2:
        dirs[m] = dirs[2 * m] ^ ((io & m) != 0)
        m //= 2
    m = 2
    while m <= bsz:
        d = dirs[m]
        s = m // 2
        while s >= 1:
            up = (io & s) != 0
            k, i, pay = _ce(k, i, pay, s, up, d)
            s //= 2
        m *= 2
    return k, i, pay


def _merge_cleanup(k, i, pay, dirmask):
    for s in (32, 16, 8, 4, 2, 1):
        up = (_lane_iota(k.shape[1]) & s) != 0
        k, i, pay = _ce(k, i, pay, s, up, dirmask)
    return k, i, pay


def _top64(k, i):
    W = k.shape[1]
    fd = _lane_iota(W) >= (W // 2)
    k, i, _ = _sort_blocks(k, i, [], 64, fd)
    while W > TOPW:
        H = W // 2
        ka, kb = k[:, :H], k[:, H:]
        ia, ib = i[:, :H], i[:, H:]
        cmp = (ka > kb) | ((ka == kb) & (ia > ib))
        k = jnp.where(cmp, kb, ka)
        i = jnp.where(cmp, ib, ia)
        W = H
        if W > TOPW:
            d = _lane_iota(W) >= (W // 2)
        else:
            d = jnp.zeros((1, W), dtype=bool)
        k, i, _ = _merge_cleanup(k, i, [], d)
    return k, i


def _knn_body(q_ref, c_ref, idx_ref, *, rblk):
    q = q_ref[...]
    c = c_ref[...]
    qs = jnp.sum(q * q, axis=1, keepdims=True)
    cs = jnp.sum(c * c, axis=1, keepdims=True)
    g = jnp.dot(q.astype(jnp.bfloat16), c.T.astype(jnp.bfloat16),
                preferred_element_type=jnp.float32)
    d2 = qs + cs.T - 2.0 * g
    ii = jax.lax.broadcasted_iota(jnp.int32, (rblk, NSEG), 1)
    _, itop = _top64(d2, ii)
    idx_ref[...] = itop + pl.program_id(0) * NSEG


def _knn_top64(center, rblk=1024):
    N = center.shape[0]
    nseg = N // NSEG
    nr = NSEG // rblk
    return pl.pallas_call(
        functools.partial(_knn_body, rblk=rblk),
        grid=(nseg, nr),
        in_specs=[
            pl.BlockSpec((rblk, 3), lambda s, r: (s * nr + r, 0)),
            pl.BlockSpec((NSEG, 3), lambda s, r: (s, 0)),
        ],
        out_specs=pl.BlockSpec((rblk, TOPW), lambda s, r: (s * nr + r, 0)),
        out_shape=jax.ShapeDtypeStruct((N, TOPW), jnp.int32),
        interpret=_INTERPRET,
    )(center, center)


def _sc_gather(xt, yt, zt, idx2d):
    NR, RW = idx2d.shape
    NW = 32
    rpw = NR // NW
    mesh = plsc.VectorSubcoreMesh(core_axis_name="c", subcore_axis_name="s")
    o = jax.ShapeDtypeStruct((NR, RW), jnp.float32)

    @functools.partial(
        pl.kernel, mesh=mesh,
        out_type=[o, o, o],
        scratch_types=[
            pltpu.VMEM((rpw, RW), jnp.int32),
            pltpu.VMEM((rpw, RW), jnp.float32),
            pltpu.VMEM((rpw, RW), jnp.float32),
            pltpu.VMEM((rpw, RW), jnp.float32),
            pltpu.SemaphoreType.DMA,
        ],
    )
    def gk(xt_h, yt_h, zt_h, idx_h, xo_h, yo_h, zo_h,
           idx_v, xb, yb, zb, sem):
        wid = jax.lax.axis_index("s") * 2 + jax.lax.axis_index("c")
        base = wid * rpw
        pltpu.sync_copy(idx_h.at[pl.ds(base, rpw)], idx_v)

        def body(j, _):
            row = idx_v.at[j]
            pltpu.async_copy(xt_h.at[row], xb.at[j], sem).wait()
            pltpu.async_copy(yt_h.at[row], yb.at[j], sem).wait()
            pltpu.async_copy(zt_h.at[row], zb.at[j], sem).wait()
            return 0

        jax.lax.fori_loop(0, rpw, body, 0)
        pltpu.sync_copy(xb, xo_h.at[pl.ds(base, rpw)])
        pltpu.sync_copy(yb, yo_h.at[pl.ds(base, rpw)])
        pltpu.sync_copy(zb, zo_h.at[pl.ds(base, rpw)])

    return gk(xt, yt, zt, idx2d)


_C = np.float32(0.7071)
_CB = np.float32(np.array(0.7071, dtype=jnp.bfloat16))


def _bf(x):
    return x.astype(jnp.bfloat16).astype(jnp.float32)


def _rot3k(dx, dy, dz):
    dx, dy, dz = _bf(dx), _bf(dy), _bf(dz)
    rx = (0.5 * dx + _CB * dy) - 0.5 * dz
    ry = (-0.5 * dx + _CB * dy) + 0.5 * dz
    rz = _CB * dx + _CB * dz
    return rx, ry, rz


def _atan2(y, x):
    ax, ay = jnp.abs(x), jnp.abs(y)
    sw = ay > ax
    num = jnp.where(sw, ax, ay)
    den = jnp.where(sw, ay, ax)
    r = num / jnp.where(den == 0, 1.0, den)
    z = r * r
    a = ((((0.0805374449538 * z - 0.138776856032) * z + 0.199777106478) * z
          - 0.333329491539) * z) * r + r
    a = jnp.where(sw, np.float32(PI / 2) - a, a)
    xneg = jax.lax.bitcast_convert_type(x, jnp.int32) < 0
    a = jnp.where(xneg, np.float32(PI) - a, a)
    yneg = jax.lax.bitcast_convert_type(y, jnp.int32) < 0
    return jnp.where(yneg, -a, a)


def _pseudo_angle(ry, rx):
    sab = jnp.abs(rx) + jnp.abs(ry)
    core = 1.0 - rx / jnp.where(sab == 0, 1.0, sab)
    yneg = jax.lax.bitcast_convert_type(ry, jnp.int32) < 0
    p = jnp.where(yneg, -core, core)
    return jnp.where(sab == 0, 0.0, p)


def _geom_body(xg_ref, yg_ref, zg_ref, c_ref, feat_ref, mom_ref, *, rblk):
    R = rblk
    INF = jnp.float32(np.inf)
    lane = jax.lax.broadcasted_iota(jnp.int32, (1, TOPW), 1)
    dx = xg_ref[...] - c_ref[:, 0:1]
    dy = yg_ref[...] - c_ref[:, 1:2]
    dz = zg_ref[...] - c_ref[:, 2:3]
    rx, ry, rz = _rot3k(dx, dy, dz)
    rho = jnp.sqrt(rx * rx + ry * ry + rz * rz)
    key = jnp.where(lane < KK, rho, INF)
    pos = jax.lax.broadcasted_iota(jnp.int32, (R, TOPW), 1)
    asc = jnp.zeros((1, TOPW), dtype=bool)
    _, _, (dx, dy, dz) = _sort_blocks(key, pos, [dx, dy, dz], TOPW, asc)

    sparts, nparts = [], []
    for off, m, M in ((0, 9, 16), (9, 18, 32), (27, 27, 32)):
        pads = M - m
        px = jnp.concatenate([dx[:, off:off + m], jnp.zeros((R, pads), jnp.float32)], 1)
        py = jnp.concatenate([dy[:, off:off + m], jnp.zeros((R, pads), jnp.float32)], 1)
        pz = jnp.concatenate([dz[:, off:off + m], jnp.zeros((R, pads), jnp.float32)], 1)
        prx, pry, _ = _rot3k(px, py, pz)
        pa = _pseudo_angle(pry, prx)
        pl_lane = _lane_iota(M)
        k2 = jnp.where(pl_lane < m, pa, INF)
        p2 = jax.lax.broadcasted_iota(jnp.int32, (R, M), 1)
        ascM = jnp.zeros((1, M), dtype=bool)
        _, _, (px, py, pz) = _sort_blocks(k2, p2, [px, py, pz], M, ascM)
        wrap = pl_lane == (m - 1)
        nx = jnp.where(wrap, px[:, 0:1], _rotl(px, 1))
        ny = jnp.where(wrap, py[:, 0:1], _rotl(py, 1))
        nz = jnp.where(wrap, pz[:, 0:1], _rotl(pz, 1))
        sparts.append((px[:, :m], py[:, :m], pz[:, :m]))
        nparts.append((nx[:, :m], ny[:, :m], nz[:, :m]))

    zpad = jnp.zeros((R, TOPW - KK), jnp.float32)
    sdx = jnp.concatenate([sparts[0][0], sparts[1][0], sparts[2][0], zpad], 1)
    sdy = jnp.concatenate([sparts[0][1], sparts[1][1], sparts[2][1], zpad], 1)
    sdz = jnp.concatenate([sparts[0][2], sparts[1][2], sparts[2][2], zpad], 1)
    ndx = jnp.concatenate([nparts[0][0], nparts[1][0], nparts[2][0], zpad], 1)
    ndy = jnp.concatenate([nparts[0][1], nparts[1][1], nparts[2][1], zpad], 1)
    ndz = jnp.concatenate([nparts[0][2], nparts[1][2], nparts[2][2], zpad], 1)

    nx = sdy * ndz - sdz * ndy
    ny = sdz * ndx - sdx * ndz
    nz = sdx * ndy - sdy * ndx
    nn = jnp.sqrt(nx * nx + ny * ny + nz * nz)
    bad = nn == 0
    inv = 1.0 / jnp.where(bad, 1.0, nn)
    ux, uy, uz = nx * inv, ny * inv, nz * inv
    pm = jnp.where(ux[:, 0:1] > 0, 1.0, -1.0).astype(jnp.float32)
    ux, uy, uz = ux * pm, uy * pm, uz * pm

    gcx = (sdx + ndx) / 3.0
    gcy = (sdy + ndy) / 3.0
    gcz = (sdz + ndz) / 3.0
    rc = jnp.sqrt(gcx * gcx + gcy * gcy + gcz * gcz)
    rsafe = jnp.where(rc == 0, 1.0, rc)
    tt = jnp.clip(gcz / rsafe, -1.0, 1.0)
    theta = _atan2(jnp.sqrt(jnp.maximum(1.0 - tt * tt, 0.0)), tt)
    theta = jnp.where(rc == 0, 0.0, theta) * np.float32(1.0 / PI)
    phic = _atan2(gcy, gcx) * np.float32(1.0 / (2 * PI)) + 0.5
    gpos = (ux * gcx + uy * gcy + uz * gcz) / np.float32(np.sqrt(3.0))

    cand = jnp.where((~bad) & (lane < KK), pos, 999)
    fi = jnp.min(cand, axis=1, keepdims=True)
    fi = jnp.where(fi == 999, 0, fi)
    sel = (pos == fi).astype(jnp.float32)

    def firstv(v):
        return jnp.sum(v * sel, axis=1, keepdims=True)

    ux = jnp.where(bad, firstv(ux), ux)
    uy = jnp.where(bad, firstv(uy), uy)
    uz = jnp.where(bad, firstv(uz), uz)
    gpos = jnp.where(bad, firstv(gpos), gpos)
    gcxf = jnp.where(bad, firstv(gcx), gcx)
    gcyf = jnp.where(bad, firstv(gcy), gcy)
    gczf = jnp.where(bad, firstv(gcz), gcz)

    valid = lane < KK
    feats = [rc, theta, phic, ux, uy, uz, gpos, gcxf, gcyf, gczf]
    feats = [jnp.where(valid, f, 0.0) for f in feats]
    for cix in range(10):
        feat_ref[cix] = feats[cix]

    rio = jax.lax.broadcasted_iota(jnp.int32, (16, 128), 0)
    cio = jax.lax.broadcasted_iota(jnp.int32, (16, 128), 1)
    mp = jnp.zeros((16, 128), jnp.float32)
    for c in range(10):
        sc_ = jnp.sum(feats[c])
        mp = mp + jnp.where((rio == 10) & (cio == c), sc_, 0.0)
        for d in range(c, 10):
            v = jnp.sum(feats[c] * feats[d])
            msk = (rio == c) & (cio == d)
            if d != c:
                msk = msk | ((rio == d) & (cio == c))
            mp = mp + jnp.where(msk, v, 0.0)

    @pl.when(pl.program_id(0) == 0)
    def _():
        mom_ref[...] = jnp.zeros((16, 128), jnp.float32)

    mom_ref[...] += mp


def _geometry(xg, yg, zg, center, rblk=256):
    N = center.shape[0]
    return pl.pallas_call(
        functools.partial(_geom_body, rblk=rblk),
        grid=(N // rblk,),
        in_specs=[
            pl.BlockSpec((rblk, TOPW), lambda i: (i, 0)),
            pl.BlockSpec((rblk, TOPW), lambda i: (i, 0)),
            pl.BlockSpec((rblk, TOPW), lambda i: (i, 0)),
            pl.BlockSpec((rblk, 3), lambda i: (i, 0)),
        ],
        out_specs=[
            pl.BlockSpec((10, rblk, TOPW), lambda i: (0, i, 0)),
            pl.BlockSpec((16, 128), lambda i: (0, 0)),
        ],
        out_shape=[
            jax.ShapeDtypeStruct((10, N, TOPW), jnp.float32),
            jax.ShapeDtypeStruct((16, 128), jnp.float32),
        ],
        interpret=_INTERPRET,
    )(xg, yg, zg, center)


def _mlp_body(f_ref, w1_ref, b_ref, w2_ref, b2_ref, out_ref, *, rblk):
    R = rblk
    lane = jax.lax.broadcasted_iota(jnp.int32, (1, TOPW), 1)
    valid = lane < KK
    colio = jax.lax.broadcasted_iota(jnp.int32, (1, 64), 1)
    F = [f_ref[c] for c in range(10)]
    z = jnp.zeros((R, 64), jnp.float32)
    for o in range(64):
        y = jnp.full((R, TOPW), b_ref[0, o], jnp.float32)
        for c in range(10):
            y = y + w1_ref[c, o] * F[c]
        y = jnp.maximum(y, 0.0)
        zo = jnp.sum(jnp.where(valid, y, 0.0), axis=1, keepdims=True)
        z = z + jnp.where(colio == o, zo, 0.0)
    out_ref[...] = (jnp.dot(z, w2_ref[...], preferred_element_type=jnp.float32)
                    + b2_ref[...])


def _mlp(f_all, w1s, bias_eff, W2T, b2eff, rblk=256):
    N = f_all.shape[1]
    return pl.pallas_call(
        functools.partial(_mlp_body, rblk=rblk),
        grid=(N // rblk,),
        in_specs=[
            pl.BlockSpec((10, rblk, TOPW), lambda i: (0, i, 0)),
            pl.BlockSpec((16, 64), lambda i: (0, 0)),
            pl.BlockSpec((1, 64), lambda i: (0, 0)),
            pl.BlockSpec((64, 64), lambda i: (0, 0)),
            pl.BlockSpec((1, 64), lambda i: (0, 0)),
        ],
        out_specs=pl.BlockSpec((rblk, 64), lambda i: (i, 0)),
        out_shape=jax.ShapeDtypeStruct((N, 64), jnp.float32),
        interpret=_INTERPRET,
    )(f_all, w1s, bias_eff, W2T, b2eff)


def _gather_xyz(center, idx):
    N = center.shape[0]
    xt = center[:, 0]
    yt = center[:, 1]
    zt = center[:, 2]
    xg, yg, zg = _sc_gather(xt, yt, zt, idx.reshape(-1, 128))
    return xg.reshape(N, TOPW), yg.reshape(N, TOPW), zg.reshape(N, TOPW)


def kernel(center, offset, W1, b1, gamma, beta, W2, b2):
    N = center.shape[0]
    idx = _knn_top64(center)
    xg, yg, zg = _gather_xyz(center, idx)
    f_all, mom = _geometry(xg, yg, zg, center)

    L = N * KK
    Mm = mom[:10, :10]
    s = mom[10, :10]
    w1s_raw = W1 @ s
    mean = w1s_raw / L + b1
    ey2 = jnp.einsum('oc,cd,od->o', W1, Mm, W1) / L + 2.0 * b1 * (w1s_raw / L) + b1 ** 2
    var = ey2 - mean ** 2
    scale = gamma * jax.lax.rsqrt(var + 1e-5)
    w1s = jnp.pad(W1.T * scale[None, :], ((0, 6), (0, 0)))
    bias_eff = ((b1 - mean) * scale + beta)[None, :]
    return _mlp(f_all, w1s, bias_eff, W2.T, (KK * b2)[None, :])
